# Initial kernel scaffold; baseline (speedup 1.0000x reference)
import jax
import jax.numpy as jnp
from jax.experimental import pallas as pl


def kernel(x, W_qkv, b_qkv, W_out, b_out):
    raise NotImplementedError("write your pallas kernel here")


# speedup vs baseline: 14.7943x; 14.7943x over previous
import math

import jax
import jax.numpy as jnp
from jax.experimental import pallas as pl
from jax.experimental.pallas import tpu as pltpu

_B, _S, _D, _H, _TOPK = 1, 2048, 1024, 16, 64
_DH = _D // _H
_SCALE = 1.0 / math.sqrt(_DH)
_INT_MIN = -2147483648


def _fused_attn_kernel(x_ref, wqkv_ref, bqkv_ref, wout_ref, bout_ref, out_ref):
    h = pl.program_id(0)
    x = x_ref[...]
    w = wqkv_ref[0]
    b = bqkv_ref[0]
    qkv = jnp.dot(x, w.T, preferred_element_type=jnp.float32) + b
    q = qkv[:, :_DH]
    k = qkv[:, _DH:2 * _DH]
    v = qkv[:, 2 * _DH:]

    logits = jnp.dot(q, k.T, preferred_element_type=jnp.float32) * _SCALE

    ikey = jax.lax.bitcast_convert_type(logits, jnp.int32)
    key = jnp.where(ikey >= 0, ikey, ikey ^ jnp.int32(0x7FFFFFFF))

    cnt0 = jnp.sum((key >= 0).astype(jnp.int32), axis=1, keepdims=True)
    t = jnp.where(cnt0 >= _TOPK, jnp.int32(0), jnp.int32(_INT_MIN))
    for bit in range(30, -1, -1):
        cand = t | jnp.int32(1 << bit)
        cnt = jnp.sum((key >= cand).astype(jnp.int32), axis=1, keepdims=True)
        t = jnp.where(cnt >= _TOPK, cand, t)

    mkey = jnp.max(key, axis=1, keepdims=True)
    ikey2 = jnp.where(key >= 0, key, key ^ jnp.int32(0x7FFFFFFF))
    logits2 = jax.lax.bitcast_convert_type(ikey2, jnp.float32)
    m = jax.lax.bitcast_convert_type(
        jnp.where(mkey >= 0, mkey, mkey ^ jnp.int32(0x7FFFFFFF)), jnp.float32)
    wexp = jnp.where(key >= t, jnp.exp(logits2 - m), 0.0)
    denom = jnp.sum(wexp, axis=1, keepdims=True)
    wexp = wexp / denom
    attn = jnp.dot(wexp, v, preferred_element_type=jnp.float32)

    wo = wout_ref[0]
    contrib = jnp.dot(attn, wo, preferred_element_type=jnp.float32)

    @pl.when(h == 0)
    def _():
        out_ref[...] = x + bout_ref[...] + contrib

    @pl.when(h != 0)
    def _():
        out_ref[...] += contrib


def kernel(x, W_qkv, b_qkv, W_out, b_out):
    b, s, d = x.shape
    x2 = x.reshape(s, d)
    wqkv = W_qkv.reshape(_H, 3 * _DH, d)
    bqkv = b_qkv.reshape(_H, 1, 3 * _DH)
    wout_t = W_out.T.reshape(_H, _DH, d)
    bout = b_out.reshape(1, d)

    out = pl.pallas_call(
        _fused_attn_kernel,
        grid=(_H,),
        in_specs=[
            pl.BlockSpec((s, d), lambda h: (0, 0)),
            pl.BlockSpec((1, 3 * _DH, d), lambda h: (h, 0, 0)),
            pl.BlockSpec((1, 1, 3 * _DH), lambda h: (h, 0, 0)),
            pl.BlockSpec((1, _DH, d), lambda h: (h, 0, 0)),
            pl.BlockSpec((1, d), lambda h: (0, 0)),
        ],
        out_specs=pl.BlockSpec((s, d), lambda h: (0, 0)),
        out_shape=jax.ShapeDtypeStruct((s, d), jnp.float32),
        compiler_params=pltpu.CompilerParams(
            dimension_semantics=("arbitrary",),
            vmem_limit_bytes=110 * 1024 * 1024,
        ),
    )(x2, wqkv, bqkv, wout_t, bout)
    return out.reshape(b, s, d)
